# initial kernel scaffold (unmeasured)
import jax
import jax.numpy as jnp
from jax import lax
from jax.experimental import pallas as pl
from jax.experimental.pallas import tpu as pltpu

N_DEV = 4


def kernel(x, router_W, route_idx, expert_W):
    n_tok, d_model = x.shape
    n_exp = router_W.shape[1]
    e_loc, _, d_ff = expert_W.shape

    def body(x_ref, rw_ref, idx_ref, ew_ref, out_ref, comm_ref, send_sems, recv_sems):
        my = lax.axis_index("i")
        left = lax.rem(my + N_DEV - 1, N_DEV)
        right = lax.rem(my + 1, N_DEV)

        barrier = pltpu.get_barrier_semaphore()
        for nbr in (left, right):
            pl.semaphore_signal(
                barrier, inc=1, device_id=(nbr,),
                device_id_type=pl.DeviceIdType.MESH,
            )
        pl.semaphore_wait(barrier, 2)

        scores = jnp.dot(x_ref[...], rw_ref[...], preferred_element_type=jnp.float32)
        m = jnp.max(scores, axis=-1, keepdims=True)
        p = jnp.exp(scores - m)
        p = p / jnp.sum(p, axis=-1, keepdims=True)
        cols = lax.broadcasted_iota(jnp.int32, (n_tok, n_exp), 1)
        mask = (cols == idx_ref[:, 0:1]) | (cols == idx_ref[:, 1:2])
        pm = jnp.where(mask, p, 0.0)
        gates = pm / jnp.sum(pm, axis=-1, keepdims=True)

        def compute_group(h, w_ref, is_first):
            origin = lax.rem(my - h + 2 * N_DEV, N_DEV)
            g = lax.dynamic_slice_in_dim(gates, origin * e_loc, e_loc, axis=1)
            w = w_ref[...]
            for e in range(e_loc):
                xg = x_ref[...] * g[:, e:e + 1]
                contrib = jnp.dot(xg, w[e], preferred_element_type=jnp.float32)
                if is_first and e == 0:
                    out_ref[...] = contrib
                else:
                    out_ref[...] += contrib

        rdmas = []
        for h in range(N_DEV - 1):
            src = ew_ref if h == 0 else comm_ref.at[h - 1]
            rdma = pltpu.make_async_remote_copy(
                src_ref=src,
                dst_ref=comm_ref.at[h],
                send_sem=send_sems.at[h],
                recv_sem=recv_sems.at[h],
                device_id=(right,),
                device_id_type=pl.DeviceIdType.MESH,
            )
            rdma.start()
            rdmas.append(rdma)
            compute_group(h, src, is_first=(h == 0))
            rdma.wait_recv()

        compute_group(N_DEV - 1, comm_ref.at[N_DEV - 2], is_first=False)

        for rdma in rdmas:
            rdma.wait_send()

    return pl.pallas_call(
        body,
        out_shape=jax.ShapeDtypeStruct((n_tok, d_ff), jnp.float32),
        in_specs=[
            pl.BlockSpec(memory_space=pltpu.VMEM),
            pl.BlockSpec(memory_space=pltpu.VMEM),
            pl.BlockSpec(memory_space=pltpu.VMEM),
            pl.BlockSpec(memory_space=pltpu.VMEM),
        ],
        out_specs=pl.BlockSpec(memory_space=pltpu.VMEM),
        scratch_shapes=[
            pltpu.VMEM((N_DEV - 1, e_loc, d_model, d_ff), jnp.float32),
            pltpu.SemaphoreType.DMA((N_DEV - 1,)),
            pltpu.SemaphoreType.DMA((N_DEV - 1,)),
        ],
        compiler_params=pltpu.CompilerParams(collective_id=0),
    )(x, router_W, route_idx, expert_W)


# baseline (device time: 299113 ns/iter reference)
import jax
import jax.numpy as jnp
from jax import lax
from jax.experimental import pallas as pl
from jax.experimental.pallas import tpu as pltpu

N_DEV = 4


def kernel(x, router_W, route_idx, expert_W):
    n_tok, d_model = x.shape
    n_exp = router_W.shape[1]
    e_loc, _, d_ff = expert_W.shape

    def body(x_ref, rw_ref, idx_ref, ew_ref, out_ref, comm_ref, send_sems, recv_sems):
        my = lax.axis_index("i")
        left = lax.rem(my + N_DEV - 1, N_DEV)
        right = lax.rem(my + 1, N_DEV)

        barrier = pltpu.get_barrier_semaphore()
        for nbr in (left, right):
            pl.semaphore_signal(
                barrier, inc=1, device_id=(nbr,),
                device_id_type=pl.DeviceIdType.MESH,
            )
        pl.semaphore_wait(barrier, 2)

        scores = jnp.dot(x_ref[...], rw_ref[...], preferred_element_type=jnp.float32)
        m = jnp.max(scores, axis=-1, keepdims=True)
        p = jnp.exp(scores - m)
        p = p / jnp.sum(p, axis=-1, keepdims=True)
        cols = lax.broadcasted_iota(jnp.int32, (n_tok, n_exp), 1)
        mask = (cols == idx_ref[:, 0:1]) | (cols == idx_ref[:, 1:2])
        pm = jnp.where(mask, p, 0.0)
        gates = pm / jnp.sum(pm, axis=-1, keepdims=True)

        def compute_group(h, w_ref, is_first):
            origin = lax.rem(my - h + 2 * N_DEV, N_DEV)
            w = w_ref[...]
            for e in range(e_loc):
                ge = origin * e_loc + e
                g = jnp.sum(jnp.where(cols == ge, gates, 0.0), axis=1,
                            keepdims=True)
                xg = x_ref[...] * g
                contrib = jnp.dot(xg, w[e], preferred_element_type=jnp.float32)
                if is_first and e == 0:
                    out_ref[...] = contrib
                else:
                    out_ref[...] += contrib

        rdmas = []
        for h in range(N_DEV - 1):
            src = ew_ref if h == 0 else comm_ref.at[h - 1]
            rdma = pltpu.make_async_remote_copy(
                src_ref=src,
                dst_ref=comm_ref.at[h],
                send_sem=send_sems.at[h],
                recv_sem=recv_sems.at[h],
                device_id=(right,),
                device_id_type=pl.DeviceIdType.MESH,
            )
            rdma.start()
            rdmas.append(rdma)
            compute_group(h, src, is_first=(h == 0))
            rdma.wait_recv()

        compute_group(N_DEV - 1, comm_ref.at[N_DEV - 2], is_first=False)

        for rdma in rdmas:
            rdma.wait_send()

    return pl.pallas_call(
        body,
        out_shape=jax.ShapeDtypeStruct((n_tok, d_ff), jnp.float32),
        in_specs=[
            pl.BlockSpec(memory_space=pltpu.VMEM),
            pl.BlockSpec(memory_space=pltpu.VMEM),
            pl.BlockSpec(memory_space=pltpu.VMEM),
            pl.BlockSpec(memory_space=pltpu.VMEM),
        ],
        out_specs=pl.BlockSpec(memory_space=pltpu.VMEM),
        scratch_shapes=[
            pltpu.VMEM((N_DEV - 1, e_loc, d_model, d_ff), jnp.float32),
            pltpu.SemaphoreType.DMA((N_DEV - 1,)),
            pltpu.SemaphoreType.DMA((N_DEV - 1,)),
        ],
        compiler_params=pltpu.CompilerParams(collective_id=0),
    )(x, router_W, route_idx, expert_W)


# device time: 163465 ns/iter; 1.8298x vs baseline; 1.8298x over previous
import jax
import jax.numpy as jnp
from jax import lax
from jax.experimental import pallas as pl
from jax.experimental.pallas import tpu as pltpu

N_DEV = 4


def kernel(x, router_W, route_idx, expert_W):
    n_tok, d_model = x.shape
    n_exp = router_W.shape[1]
    e_loc, _, d_ff = expert_W.shape

    def body(x_ref, rw_ref, idx_ref, ew_ref, out_ref, g0_ref, comm_ref,
             send_sems, recv_sems):
        my = lax.axis_index("i")
        left = lax.rem(my + N_DEV - 1, N_DEV)
        right = lax.rem(my + 1, N_DEV)

        barrier = pltpu.get_barrier_semaphore()
        for nbr in (left, right):
            pl.semaphore_signal(
                barrier, inc=1, device_id=(nbr,),
                device_id_type=pl.DeviceIdType.MESH,
            )
        pl.semaphore_wait(barrier, 2)

        scores = jnp.dot(x_ref[...], rw_ref[...], preferred_element_type=jnp.float32)
        m = jnp.max(scores, axis=-1, keepdims=True)
        p = jnp.exp(scores - m)
        p = p / jnp.sum(p, axis=-1, keepdims=True)
        cols = lax.broadcasted_iota(jnp.int32, (n_tok, n_exp), 1)
        mask = (cols == idx_ref[:, 0:1]) | (cols == idx_ref[:, 1:2])
        pm = jnp.where(mask, p, 0.0)
        gates = pm / jnp.sum(pm, axis=-1, keepdims=True)

        x_b = x_ref[...].astype(jnp.bfloat16)
        g0_ref[...] = ew_ref[...].astype(jnp.bfloat16)

        def compute_group(h, w_ref, is_first):
            origin = lax.rem(my - h + 2 * N_DEV, N_DEV)
            w = w_ref[...]
            for e in range(e_loc):
                ge = origin * e_loc + e
                g = jnp.sum(jnp.where(cols == ge, gates, 0.0), axis=1,
                            keepdims=True)
                y = jnp.dot(x_b, w[e], preferred_element_type=jnp.float32)
                if is_first and e == 0:
                    out_ref[...] = g * y
                else:
                    out_ref[...] += g * y

        rdmas = []
        for h in range(N_DEV - 1):
            src = g0_ref if h == 0 else comm_ref.at[h - 1]
            rdma = pltpu.make_async_remote_copy(
                src_ref=src,
                dst_ref=comm_ref.at[h],
                send_sem=send_sems.at[h],
                recv_sem=recv_sems.at[h],
                device_id=(right,),
                device_id_type=pl.DeviceIdType.MESH,
            )
            rdma.start()
            rdmas.append(rdma)
            compute_group(h, src, is_first=(h == 0))
            rdma.wait_recv()

        compute_group(N_DEV - 1, comm_ref.at[N_DEV - 2], is_first=False)

        for rdma in rdmas:
            rdma.wait_send()

    return pl.pallas_call(
        body,
        out_shape=jax.ShapeDtypeStruct((n_tok, d_ff), jnp.float32),
        in_specs=[
            pl.BlockSpec(memory_space=pltpu.VMEM),
            pl.BlockSpec(memory_space=pltpu.VMEM),
            pl.BlockSpec(memory_space=pltpu.VMEM),
            pl.BlockSpec(memory_space=pltpu.VMEM),
        ],
        out_specs=pl.BlockSpec(memory_space=pltpu.VMEM),
        scratch_shapes=[
            pltpu.VMEM((e_loc, d_model, d_ff), jnp.bfloat16),
            pltpu.VMEM((N_DEV - 1, e_loc, d_model, d_ff), jnp.bfloat16),
            pltpu.SemaphoreType.DMA((N_DEV - 1,)),
            pltpu.SemaphoreType.DMA((N_DEV - 1,)),
        ],
        compiler_params=pltpu.CompilerParams(collective_id=0),
    )(x, router_W, route_idx, expert_W)
